# baseline (speedup 1.0000x reference)
import functools

import jax
import jax.numpy as jnp
from jax import lax
from jax.experimental import pallas as pl
from jax.experimental.pallas import tpu as pltpu
from jax.experimental.pallas import tpu_sc as plsc

_NUM_CORES = 2
_NUM_SUBCORES = 16
_NW = _NUM_CORES * _NUM_SUBCORES
_L = 16
_ZPAD = 64


@functools.cache
def _build_encoder(n, c_dim):
  n_pad = -(-n // 128) * 128
  chunk = 128
  for c in (1664, 2048, 1024, 512, 256, 128):
    if n_pad % c == 0:
      chunk = c
      break
  groups = chunk // _L
  nchunks = n_pad // chunk
  kmax = -(-nchunks // _NW)

  mesh = plsc.VectorSubcoreMesh(
      core_axis_name="c", subcore_axis_name="s",
      num_cores=_NUM_CORES, num_subcores=_NUM_SUBCORES)

  @functools.partial(
      pl.kernel,
      out_type=jax.ShapeDtypeStruct((c_dim, n), jnp.float32),
      mesh=mesh,
      compiler_params=pltpu.CompilerParams(
          needs_layout_passes=False, disable_bounds_checks=True),
      scratch_types=[
          pltpu.VMEM((2 * _L,), jnp.int32),
          pltpu.VMEM((2 * _L,), jnp.float32),
      ] + [
          pltpu.VMEM((_ZPAD,), jnp.float32) for _ in range(c_dim)
      ] + [
          pltpu.VMEM((chunk,), jnp.int32),
          pltpu.VMEM((chunk,), jnp.int32),
          pltpu.VMEM((c_dim, chunk), jnp.float32),
          pltpu.VMEM((c_dim, chunk), jnp.float32),
          pltpu.SemaphoreType.DMA,
          pltpu.SemaphoreType.DMA,
          pltpu.SemaphoreType.DMA,
          pltpu.SemaphoreType.DMA,
      ],
  )
  def encode(z_hbm, zs_hbm, out_hbm, zs_i, zs_f, *rest):
    col_tabs = rest[:c_dim]
    (z_b0, z_b1, out_b0, out_b1,
     zsem0, zsem1, osem0, osem1) = rest[c_dim:]
    wid = lax.axis_index("s") * _NUM_CORES + lax.axis_index("c")
    iota = lax.iota(jnp.int32, _L)

    _OFF = 8
    for half in range(2):
      zs_i[pl.ds(half * _L, _L)] = jnp.full((_L,), 1 << 30, jnp.int32)
    pltpu.sync_copy(zs_hbm, zs_i.at[pl.ds(_OFF, c_dim)])
    for half in range(2):
      sl = pl.ds(half * _L, _L)
      zs_f[sl] = zs_i[sl].astype(jnp.float32)

    for g in range(_ZPAD // _L):
      z = g * _L + iota
      j = jnp.zeros((_L,), jnp.int32)
      for cc in range(c_dim):
        zs_cc = plsc.load_gather(
            zs_i, [jnp.full((_L,), _OFF + cc, jnp.int32)])
        j = j + jnp.where(zs_cc < z, 1, 0)
      j = jnp.minimum(j, c_dim - 1)
      lo = jnp.maximum(j - 1, 0)
      hi = j
      z_f = z.astype(jnp.float32)
      zs_lo = plsc.load_gather(zs_f, [lo + _OFF])
      zs_hi = plsc.load_gather(zs_f, [hi + _OFF])
      exact = plsc.load_gather(zs_i, [hi + _OFF]) == z
      denom = jnp.maximum(zs_hi - zs_lo, 1.0)
      w_lo = (zs_hi - z_f) / denom
      w_hi = (z_f - zs_lo) / denom
      col0 = jnp.where(exact, hi, lo)
      val0 = jnp.where(exact, 1.0, w_lo)
      val1 = jnp.where(exact, 0.0, w_hi)
      zeros_v = jnp.zeros((_L,), jnp.float32)
      sl = pl.ds(g * _L, _L)
      for t in range(c_dim):
        col_tabs[t][sl] = (jnp.where(col0 == t, val0, zeros_v)
                           + jnp.where(hi == t, val1, zeros_v))

    def compute_chunk(z_buf, out_buf):
      @plsc.parallel_loop(0, groups, unroll=2)
      def _groups(g):
        off = g * _L
        z = jnp.clip(z_buf[pl.ds(off, _L)], 0, _ZPAD - 1)
        for t in range(c_dim):
          out_buf[t, pl.ds(off, _L)] = plsc.load_gather(col_tabs[t], [z])

    bufs = ((z_b0, out_b0, zsem0, osem0), (z_b1, out_b1, zsem1, osem1))

    def start_z(k, z_buf, zsem):
      cidx = wid + k * _NW

      @pl.when(cidx < nchunks)
      def _():
        pltpu.async_copy(z_hbm.at[pl.ds(cidx * chunk, chunk)], z_buf, zsem)

    for b, (z_buf, _o, zsem, _os) in enumerate(bufs):
      start_z(b, z_buf, zsem)

    def do_pair(kk, carry):
      for b, (z_buf, out_buf, zsem, osem) in enumerate(bufs):
        k = kk * 2 + b
        cidx = wid + k * _NW

        @pl.when(cidx < nchunks)
        def _body(k=k, z_buf=z_buf, out_buf=out_buf, zsem=zsem, osem=osem):
          pltpu.make_async_copy(
              z_hbm.at[pl.ds(0, chunk)], z_buf, zsem).wait()

          @pl.when(k >= 2)
          def _drain():
            pltpu.make_async_copy(
                out_buf, out_hbm.at[:, pl.ds(0, chunk)], osem).wait()

          compute_chunk(z_buf, out_buf)
          pltpu.async_copy(
              out_buf, out_hbm.at[:, pl.ds(cidx * chunk, chunk)], osem)
          start_z(k + 2, z_buf, zsem)

      return carry

    lax.fori_loop(0, (kmax + 1) // 2, do_pair, 0, unroll=False)

    for b, (z_buf, out_buf, zsem, osem) in enumerate(bufs):
      @pl.when(wid + b * _NW < nchunks)
      def _final(z_buf=z_buf, out_buf=out_buf, osem=osem):
        pltpu.make_async_copy(
            out_buf, out_hbm.at[:, pl.ds(0, chunk)], osem).wait()

  return encode


def kernel(atomic_numbers, zs):
  n = atomic_numbers.shape[0]
  c_dim = zs.shape[0]
  encode = _build_encoder(n, c_dim)
  out_t = encode(atomic_numbers.astype(jnp.int32), zs.astype(jnp.int32))
  return out_t.T


# speedup vs baseline: 1.0128x; 1.0128x over previous
import functools

import jax
import jax.numpy as jnp
from jax import lax
from jax.experimental import pallas as pl
from jax.experimental.pallas import tpu as pltpu
from jax.experimental.pallas import tpu_sc as plsc

_NUM_CORES = 2
_NUM_SUBCORES = 16
_NW = _NUM_CORES * _NUM_SUBCORES
_L = 16
_ZPAD = 64


@functools.cache
def _build_encoder(n, c_dim):
  n_pad = -(-n // 128) * 128
  chunk = 128
  for c in (1664, 2048, 1024, 512, 256, 128):
    if n_pad % c == 0:
      chunk = c
      break
  groups = chunk // _L
  nchunks = n_pad // chunk
  kmax = -(-nchunks // _NW)

  mesh = plsc.VectorSubcoreMesh(
      core_axis_name="c", subcore_axis_name="s",
      num_cores=_NUM_CORES, num_subcores=_NUM_SUBCORES)

  @functools.partial(
      pl.kernel,
      out_type=jax.ShapeDtypeStruct((c_dim, n), jnp.float32),
      mesh=mesh,
      compiler_params=pltpu.CompilerParams(
          needs_layout_passes=False, disable_bounds_checks=True),
      scratch_types=[
          pltpu.VMEM((2 * _L,), jnp.int32),
          pltpu.VMEM((2 * _L,), jnp.float32),
      ] + [
          pltpu.VMEM((_ZPAD,), jnp.float32) for _ in range(c_dim)
      ] + [
          pltpu.VMEM((chunk,), jnp.int32),
          pltpu.VMEM((chunk,), jnp.int32),
          pltpu.VMEM((c_dim, chunk), jnp.float32),
          pltpu.VMEM((c_dim, chunk), jnp.float32),
          pltpu.SemaphoreType.DMA,
          pltpu.SemaphoreType.DMA,
          pltpu.SemaphoreType.DMA,
          pltpu.SemaphoreType.DMA,
      ],
  )
  def encode(z_hbm, zs_hbm, out_hbm, zs_i, zs_f, *rest):
    col_tabs = rest[:c_dim]
    (z_b0, z_b1, out_b0, out_b1,
     zsem0, zsem1, osem0, osem1) = rest[c_dim:]
    wid = lax.axis_index("s") * _NUM_CORES + lax.axis_index("c")
    iota = lax.iota(jnp.int32, _L)

    _OFF = 8
    for half in range(2):
      zs_i[pl.ds(half * _L, _L)] = jnp.full((_L,), 1 << 30, jnp.int32)
    pltpu.sync_copy(zs_hbm, zs_i.at[pl.ds(_OFF, c_dim)])
    for half in range(2):
      sl = pl.ds(half * _L, _L)
      zs_f[sl] = zs_i[sl].astype(jnp.float32)

    for g in range(_ZPAD // _L):
      z = g * _L + iota
      j = jnp.zeros((_L,), jnp.int32)
      for cc in range(c_dim):
        zs_cc = plsc.load_gather(
            zs_i, [jnp.full((_L,), _OFF + cc, jnp.int32)])
        j = j + jnp.where(zs_cc < z, 1, 0)
      j = jnp.minimum(j, c_dim - 1)
      lo = jnp.maximum(j - 1, 0)
      hi = j
      z_f = z.astype(jnp.float32)
      zs_lo = plsc.load_gather(zs_f, [lo + _OFF])
      zs_hi = plsc.load_gather(zs_f, [hi + _OFF])
      exact = plsc.load_gather(zs_i, [hi + _OFF]) == z
      denom = jnp.maximum(zs_hi - zs_lo, 1.0)
      w_lo = (zs_hi - z_f) / denom
      w_hi = (z_f - zs_lo) / denom
      col0 = jnp.where(exact, hi, lo)
      val0 = jnp.where(exact, 1.0, w_lo)
      val1 = jnp.where(exact, 0.0, w_hi)
      zeros_v = jnp.zeros((_L,), jnp.float32)
      sl = pl.ds(g * _L, _L)
      for t in range(c_dim):
        col_tabs[t][sl] = (jnp.where(col0 == t, val0, zeros_v)
                           + jnp.where(hi == t, val1, zeros_v))

    def compute_chunk(z_buf, out_buf):
      @plsc.parallel_loop(0, groups, unroll=4)
      def _groups(g):
        off = g * _L
        z = jnp.clip(z_buf[pl.ds(off, _L)], 0, _ZPAD - 1)
        for t in range(c_dim):
          out_buf[t, pl.ds(off, _L)] = plsc.load_gather(col_tabs[t], [z])

    bufs = ((z_b0, out_b0, zsem0, osem0), (z_b1, out_b1, zsem1, osem1))

    def start_z(k, z_buf, zsem):
      cidx = wid + k * _NW

      @pl.when(cidx < nchunks)
      def _():
        pltpu.async_copy(z_hbm.at[pl.ds(cidx * chunk, chunk)], z_buf, zsem)

    for b, (z_buf, _o, zsem, _os) in enumerate(bufs):
      start_z(b, z_buf, zsem)

    def do_pair(kk, carry):
      for b, (z_buf, out_buf, zsem, osem) in enumerate(bufs):
        k = kk * 2 + b
        cidx = wid + k * _NW

        @pl.when(cidx < nchunks)
        def _body(k=k, z_buf=z_buf, out_buf=out_buf, zsem=zsem, osem=osem):
          pltpu.make_async_copy(
              z_hbm.at[pl.ds(0, chunk)], z_buf, zsem).wait()

          @pl.when(k >= 2)
          def _drain():
            pltpu.make_async_copy(
                out_buf, out_hbm.at[:, pl.ds(0, chunk)], osem).wait()

          compute_chunk(z_buf, out_buf)
          pltpu.async_copy(
              out_buf, out_hbm.at[:, pl.ds(cidx * chunk, chunk)], osem)
          start_z(k + 2, z_buf, zsem)

      return carry

    lax.fori_loop(0, (kmax + 1) // 2, do_pair, 0, unroll=False)

    for b, (z_buf, out_buf, zsem, osem) in enumerate(bufs):
      @pl.when(wid + b * _NW < nchunks)
      def _final(z_buf=z_buf, out_buf=out_buf, osem=osem):
        pltpu.make_async_copy(
            out_buf, out_hbm.at[:, pl.ds(0, chunk)], osem).wait()

  return encode


def kernel(atomic_numbers, zs):
  n = atomic_numbers.shape[0]
  c_dim = zs.shape[0]
  encode = _build_encoder(n, c_dim)
  out_t = encode(atomic_numbers.astype(jnp.int32), zs.astype(jnp.int32))
  return out_t.T
